# baseline (speedup 1.0000x reference)
from functools import partial

import jax
import jax.numpy as jnp
from jax.experimental import pallas as pl
from jax.experimental.pallas import tpu as pltpu

_VMEM_LIMIT_BYTES = 56 * 1024 * 1024


def _pool_kernel(x_ref, p_ref, *, inv_hw):
    x = x_ref[...]
    t = jnp.sum(x, axis=2)
    p_ref[...] = jnp.sum(t, axis=1) * inv_hw


def _rescale_kernel(p_ref, w1_ref, b1_ref, w2t_ref, b2_ref, x_ref, o_ref):
    pm = p_ref[...]
    h = jax.lax.dot_general(
        w1_ref[...], pm, (((1,), (0,)), ((), ())),
        preferred_element_type=jnp.float32)
    h = jnp.maximum(h + b1_ref[...].T, 0.0)
    s = jax.lax.dot_general(
        w2t_ref[...], h, (((0,), (0,)), ((), ())),
        preferred_element_type=jnp.float32)
    s = jax.nn.sigmoid(s + b2_ref[...].T)
    o_ref[...] = x_ref[...] * s[:, None, None, :]


def kernel(x, w1, b1, w2, b2):
    N, C, H, W = x.shape
    Cr = w1.shape[0]
    inv_hw = 1.0 / (H * W)

    xt = jnp.transpose(x, (1, 2, 3, 0))
    w1r = jnp.asarray(w1, jnp.float32)
    b1r = jnp.asarray(b1, jnp.float32).reshape(1, Cr)
    w2t = jnp.transpose(jnp.asarray(w2, jnp.float32))
    b2r = jnp.asarray(b2, jnp.float32).reshape(1, C)

    cb = 16 if C % 16 == 0 else (8 if C % 8 == 0 else C)
    pooled = pl.pallas_call(
        partial(_pool_kernel, inv_hw=inv_hw),
        out_shape=jax.ShapeDtypeStruct((C, N), jnp.float32),
        grid=(C // cb,),
        in_specs=[pl.BlockSpec((cb, H, W, N), lambda c: (c, 0, 0, 0))],
        out_specs=pl.BlockSpec((cb, N), lambda c: (c, 0)),
        compiler_params=pltpu.CompilerParams(
            dimension_semantics=("parallel",),
            vmem_limit_bytes=_VMEM_LIMIT_BYTES),
    )(xt)

    hb = max(h for h in (7, 4, 2, 1) if H % h == 0)
    out_t = pl.pallas_call(
        _rescale_kernel,
        out_shape=jax.ShapeDtypeStruct((C, H, W, N), x.dtype),
        grid=(H // hb,),
        in_specs=[
            pl.BlockSpec((C, N), lambda h: (0, 0)),
            pl.BlockSpec((Cr, C), lambda h: (0, 0)),
            pl.BlockSpec((1, Cr), lambda h: (0, 0)),
            pl.BlockSpec((Cr, C), lambda h: (0, 0)),
            pl.BlockSpec((1, C), lambda h: (0, 0)),
            pl.BlockSpec((C, hb, W, N), lambda h: (0, h, 0, 0)),
        ],
        out_specs=pl.BlockSpec((C, hb, W, N), lambda h: (0, h, 0, 0)),
        compiler_params=pltpu.CompilerParams(
            dimension_semantics=("parallel",),
            vmem_limit_bytes=_VMEM_LIMIT_BYTES),
    )(pooled, w1r, b1r, w2t, b2r, xt)

    return jnp.transpose(out_t, (3, 0, 1, 2))


# speedup vs baseline: 1.0115x; 1.0115x over previous
from functools import partial

import jax
import jax.numpy as jnp
from jax.experimental import pallas as pl
from jax.experimental.pallas import tpu as pltpu

_VMEM_LIMIT_BYTES = 56 * 1024 * 1024


def _pool_kernel(x_ref, p_ref, *, inv_hw):
    x = x_ref[...]
    t = jnp.sum(x, axis=2)
    p_ref[...] = jnp.sum(t, axis=1) * inv_hw


def _rescale_kernel(p_ref, w1_ref, b1_ref, w2t_ref, b2_ref, x_ref, o_ref):
    pm = p_ref[...]
    h = jax.lax.dot_general(
        w1_ref[...], pm, (((1,), (0,)), ((), ())),
        preferred_element_type=jnp.float32)
    h = jnp.maximum(h + b1_ref[...].T, 0.0)
    s = jax.lax.dot_general(
        w2t_ref[...], h, (((0,), (0,)), ((), ())),
        preferred_element_type=jnp.float32)
    s = jax.nn.sigmoid(s + b2_ref[...].T)
    o_ref[...] = x_ref[...] * s[:, None, None, :]


def kernel(x, w1, b1, w2, b2):
    N, C, H, W = x.shape
    Cr = w1.shape[0]
    inv_hw = 1.0 / (H * W)

    xt = jnp.transpose(x, (1, 2, 3, 0))
    w1r = jnp.asarray(w1, jnp.float32)
    b1r = jnp.asarray(b1, jnp.float32).reshape(1, Cr)
    w2t = jnp.transpose(jnp.asarray(w2, jnp.float32))
    b2r = jnp.asarray(b2, jnp.float32).reshape(1, C)

    cb = 8 if C % 8 == 0 else C
    pooled = pl.pallas_call(
        partial(_pool_kernel, inv_hw=inv_hw),
        out_shape=jax.ShapeDtypeStruct((C, N), jnp.float32),
        grid=(C // cb,),
        in_specs=[pl.BlockSpec((cb, H, W, N), lambda c: (c, 0, 0, 0))],
        out_specs=pl.BlockSpec((cb, N), lambda c: (c, 0)),
        compiler_params=pltpu.CompilerParams(
            dimension_semantics=("parallel",),
            vmem_limit_bytes=_VMEM_LIMIT_BYTES),
    )(xt)

    hb = max(h for h in (7, 4, 2, 1) if H % h == 0)
    out_t = pl.pallas_call(
        _rescale_kernel,
        out_shape=jax.ShapeDtypeStruct((C, H, W, N), x.dtype),
        grid=(H // hb,),
        in_specs=[
            pl.BlockSpec((C, N), lambda h: (0, 0)),
            pl.BlockSpec((Cr, C), lambda h: (0, 0)),
            pl.BlockSpec((1, Cr), lambda h: (0, 0)),
            pl.BlockSpec((Cr, C), lambda h: (0, 0)),
            pl.BlockSpec((1, C), lambda h: (0, 0)),
            pl.BlockSpec((C, hb, W, N), lambda h: (0, h, 0, 0)),
        ],
        out_specs=pl.BlockSpec((C, hb, W, N), lambda h: (0, h, 0, 0)),
        compiler_params=pltpu.CompilerParams(
            dimension_semantics=("parallel",),
            vmem_limit_bytes=_VMEM_LIMIT_BYTES),
    )(pooled, w1r, b1r, w2t, b2r, xt)

    return jnp.transpose(out_t, (3, 0, 1, 2))
